# baseline (speedup 1.0000x reference)
import jax
import jax.numpy as jnp
from jax.experimental import pallas as pl

_S0, _S1, _D = 64, 64, 3
_BLOCK_B = 512


def _dist_kernel(x_ref, g_ref, o_ref):
    acc = None
    for k in range(_D):
        diff = x_ref[:, k : k + 1] - g_ref[k : k + 1, :]
        sq = diff * diff
        acc = sq if acc is None else acc + sq
    o_ref[...] = jnp.sqrt(acc)


def kernel(x, grid):
    b = x.shape[0]
    n = _S0 * _S1
    g = grid.reshape(n, _D).T
    out = pl.pallas_call(
        _dist_kernel,
        grid=(b // _BLOCK_B,),
        in_specs=[
            pl.BlockSpec((_BLOCK_B, _D), lambda i: (i, 0)),
            pl.BlockSpec((_D, n), lambda i: (0, 0)),
        ],
        out_specs=pl.BlockSpec((_BLOCK_B, n), lambda i: (i, 0)),
        out_shape=jax.ShapeDtypeStruct((b, n), jnp.float32),
    )(x, g)
    return out.reshape(b, _S0, _S1)
